# baseline (speedup 1.0000x reference)
import functools

import jax
import jax.numpy as jnp
from jax import lax
from jax.experimental import pallas as pl
from jax.experimental.pallas import tpu as pltpu
from jax.experimental.pallas import tpu_sc as plsc

N, E, D = 10000, 320000, 128
NC, NS, L = 2, 16, 16
NW = NC * NS
CH = 88
NCHP = 114
SLOTS = NCHP * CH
NPAD = 10240
TRASH_N = 224
RPT = NPAD // NS
IDP = 1024
HALF = NPAD // 2
TR_E = 256
SPE = HALF + TR_E
RPT_E = HALF // NS
WB0 = HALF // NS
WB1 = 304
WB1L = (N - HALF) - (NS - 1) * WB1
CROWS = IDP // NS
NBUF = 4
NI = 8

_mesh = functools.partial(
    plsc.VectorSubcoreMesh, core_axis_name="c", subcore_axis_name="s")


GIDS = IDP // NS


@functools.partial(
    pl.kernel,
    out_type=(jax.ShapeDtypeStruct((NC, NPAD, D), jnp.float32),
              jax.ShapeDtypeStruct((NC, IDP, D), jnp.float32)),
    mesh=_mesh(),
    scratch_types=[
        pltpu.VMEM((NI, 2, CH), jnp.int32),
        pltpu.VMEM((NBUF, CH, D), jnp.float32),
        pltpu.VMEM_SHARED((NPAD, D), jnp.float32),
        [pltpu.SemaphoreType.DMA] * NI,
        [pltpu.SemaphoreType.DMA] * NBUF,
        [pltpu.SemaphoreType.DMA] * NBUF,
    ],
)
def _edge_kernel(x_hbm, srcw_hbm, dstw_hbm, zeros_hbm, ids_hbm,
                 out_hbm, hid_hbm, idx_v, rows_v, agg_sp, isem, gsem, ssem):
    c = lax.axis_index("c")
    s = lax.axis_index("s")
    wid = s * NC + c
    r0 = s * RPT

    @pl.when(c == 0)
    def _():
        @pl.when(s < NS - 1)
        def _():
            pltpu.sync_copy(x_hbm.at[pl.ds(r0, RPT)],
                            agg_sp.at[pl.ds(r0, RPT)])

        @pl.when(s == NS - 1)
        def _():
            lo = (NS - 1) * RPT
            pltpu.sync_copy(x_hbm.at[pl.ds(lo, N - lo)],
                            agg_sp.at[pl.ds(lo, N - lo)])
            pltpu.sync_copy(zeros_hbm.at[pl.ds(N, NPAD - N)],
                            agg_sp.at[pl.ds(N, NPAD - N)])

    @pl.when(c != 0)
    def _():
        pltpu.sync_copy(zeros_hbm.at[pl.ds(r0, RPT)],
                        agg_sp.at[pl.ds(r0, RPT)])

    id_ = [None] * NCHP
    gd = [None] * NCHP
    sd = [None] * NCHP

    def _istart(t):
        d1 = pltpu.async_copy(
            srcw_hbm.at[wid].at[t], idx_v.at[t % NI].at[0], isem[t % NI])
        d2 = pltpu.async_copy(
            dstw_hbm.at[wid].at[t], idx_v.at[t % NI].at[1], isem[t % NI])
        id_[t] = (d1, d2)

    def _scatter(q):
        gd[q].wait()
        sd[q] = pltpu.async_copy(
            rows_v.at[q % NBUF], agg_sp.at[idx_v.at[q % NI].at[1]],
            ssem[q % NBUF], add=True)

    for t in range(NI):
        _istart(t)
    plsc.subcore_barrier()
    for j in range(NCHP):
        b = j % NBUF
        if j >= NBUF:
            sd[j - NBUF].wait()
            if j - NBUF + NI < NCHP:
                _istart(j - NBUF + NI)
        id_[j][0].wait()
        id_[j][1].wait()
        gd[j] = pltpu.async_copy(
            x_hbm.at[idx_v.at[j % NI].at[0]], rows_v.at[b], gsem[b])
        if j - (NBUF - 1) >= 0:
            _scatter(j - (NBUF - 1))
    for q in range(NCHP - (NBUF - 1), NCHP):
        _scatter(q)
    for j in range(NCHP - NBUF, NCHP):
        sd[j].wait()
    plsc.subcore_barrier()
    wb = pltpu.async_copy(agg_sp.at[pl.ds(r0, RPT)],
                          out_hbm.at[c].at[pl.ds(r0, RPT)], ssem[0])
    ib = s * GIDS
    ids_slot = idx_v.at[0].at[0].at[pl.ds(0, GIDS)]
    grows = rows_v.at[0].at[pl.ds(0, GIDS)]
    pltpu.sync_copy(ids_hbm.at[pl.ds(ib, GIDS)], ids_slot)
    pltpu.async_copy(agg_sp.at[ids_slot], grows, gsem[0]).wait()
    pltpu.sync_copy(grows, hid_hbm.at[c].at[pl.ds(ib, GIDS)])
    wb.wait()


BLK = 512
GMAIN = NPAD // BLK
GRID2 = GMAIN + IDP // BLK


def _tc_body(p0_ref, p1_ref, h0_ref, h1_ref, w1_ref, b1_ref, w2_ref, b2_ref,
             wi1_ref, bi1_ref, wi2_ref, bi2_ref, o_ref, y_ref):
    i = pl.program_id(0)
    main = i < GMAIN
    h = jnp.where(main, p0_ref[...] + p1_ref[...], h0_ref[...] + h1_ref[...])
    wa = jnp.where(main, w1_ref[...], wi1_ref[...])
    ba = jnp.where(main, b1_ref[...], bi1_ref[...])
    wb = jnp.where(main, w2_ref[...], wi2_ref[...])
    bb = jnp.where(main, b2_ref[...], bi2_ref[...])
    a = jnp.maximum(jnp.dot(h, wa, preferred_element_type=jnp.float32) + ba,
                    0.0)
    o = jnp.dot(a, wb, preferred_element_type=jnp.float32) + bb

    @pl.when(main)
    def _():
        o_ref[...] = o

    @pl.when(jnp.logical_not(main))
    def _():
        y_ref[...] = o


def _tc_mlps(p0, p1, h0, h1, w1t, b1, w2t, b2, wi1t, bi1, wi2t, bi2):
    p_spec = pl.BlockSpec((BLK, D), lambda i: (jnp.minimum(i, GMAIN - 1), 0))
    h_spec = pl.BlockSpec((BLK, D), lambda i: (jnp.maximum(i - GMAIN, 0), 0))
    full_spec = pl.BlockSpec((D, D), lambda i: (0, 0))
    bias_spec = pl.BlockSpec((1, D), lambda i: (0, 0))
    return pl.pallas_call(
        _tc_body,
        grid=(GRID2,),
        in_specs=[p_spec, p_spec, h_spec, h_spec,
                  full_spec, bias_spec, full_spec, bias_spec,
                  full_spec, bias_spec, full_spec, bias_spec],
        out_specs=[p_spec, h_spec],
        out_shape=[jax.ShapeDtypeStruct((NPAD, D), jnp.float32),
                   jax.ShapeDtypeStruct((IDP, D), jnp.float32)],
    )(p0, p1, h0, h1, w1t, b1, w2t, b2, wi1t, bi1, wi2t, bi2)


@functools.partial(
    pl.kernel,
    out_type=jax.ShapeDtypeStruct((N, D), jnp.float32),
    mesh=_mesh(),
    scratch_types=[
        pltpu.VMEM((CROWS,), jnp.int32),
        pltpu.VMEM((1, CROWS), jnp.int32),
        pltpu.VMEM((CROWS, D), jnp.float32),
        pltpu.VMEM_SHARED((SPE, D), jnp.float32),
    ],
)
def _scatter_kernel(out_main_hbm, ids_hbm, yid_hbm, final_hbm,
                    ids_v, idx2_v, yrows_v, outsp):
    c = lax.axis_index("c")
    s = lax.axis_index("s")
    base_row = c * HALF

    pltpu.sync_copy(out_main_hbm.at[pl.ds(base_row + s * RPT_E, RPT_E)],
                    outsp.at[pl.ds(s * RPT_E, RPT_E)])

    ib = s * CROWS
    pltpu.sync_copy(ids_hbm.at[pl.ds(ib, CROWS)], ids_v)
    for k in range(CROWS // L):
        iv = ids_v[pl.ds(k * L, L)]
        lv = iv - base_row
        oor = (lv < 0) | (lv >= HALF)
        tv = HALF + lax.rem(iv, TR_E)
        idx2_v[0, pl.ds(k * L, L)] = jnp.where(oor, tv, lv)
    pltpu.sync_copy(yid_hbm.at[pl.ds(ib, CROWS)], yrows_v)
    plsc.subcore_barrier()
    pltpu.sync_copy(yrows_v, outsp.at[idx2_v.at[0]], add=True)
    plsc.subcore_barrier()

    @pl.when(c == 0)
    def _():
        pltpu.sync_copy(outsp.at[pl.ds(s * WB0, WB0)],
                        final_hbm.at[pl.ds(s * WB0, WB0)])

    @pl.when((c != 0) & (s < NS - 1))
    def _():
        pltpu.sync_copy(outsp.at[pl.ds(s * WB1, WB1)],
                        final_hbm.at[pl.ds(HALF + s * WB1, WB1)])

    @pl.when((c != 0) & (s == NS - 1))
    def _():
        lo = (NS - 1) * WB1
        pltpu.sync_copy(outsp.at[pl.ds(lo, WB1L)],
                        final_hbm.at[pl.ds(HALF + lo, WB1L)])


def kernel(x, edge_index, node_id, W1, b1, W2, b2, Wi1, bi1, Wi2, bi2):
    zeros = jnp.zeros((NPAD, D), jnp.float32)
    npad_e = SLOTS * NW - E
    pad_i = jnp.arange(npad_e, dtype=jnp.int32)
    src_a = jnp.concatenate([edge_index[0], pad_i % N])
    dst_a = jnp.concatenate([edge_index[1], N + pad_i % TRASH_N])
    dst_a = jnp.where(src_a == dst_a, N + dst_a % TRASH_N, dst_a)
    srcw = src_a.reshape(NW, NCHP, CH)
    dstw = dst_a.reshape(NW, NCHP, CH)
    idp = jnp.arange(IDP - node_id.shape[0], dtype=jnp.int32)
    ids = jnp.concatenate([node_id, N + idp % TRASH_N])
    p, hid = _edge_kernel(x, srcw, dstw, zeros, ids)
    out_main, yid = _tc_mlps(p[0], p[1], hid[0], hid[1],
                             W1.T, b1[None, :], W2.T, b2[None, :],
                             Wi1.T, bi1[None, :], Wi2.T, bi2[None, :])
    return _scatter_kernel(out_main, ids, yid)


# speedup vs baseline: 1.0103x; 1.0103x over previous
import functools

import jax
import jax.numpy as jnp
from jax import lax
from jax.experimental import pallas as pl
from jax.experimental.pallas import tpu as pltpu
from jax.experimental.pallas import tpu_sc as plsc

N, E, D = 10000, 320000, 128
NC, NS, L = 2, 16, 16
NW = NC * NS
CH = 112
NCHP = 90
EPW = E // NW
TAIL = EPW - (NCHP - 1) * CH
NPAD = 10240
TRASH_N = 224
RPT = NPAD // NS
IDP = 1024
HALF = NPAD // 2
TR_E = 256
SPE = HALF + TR_E
RPT_E = HALF // NS
WB0 = HALF // NS
WB1 = 304
WB1L = (N - HALF) - (NS - 1) * WB1
CROWS = IDP // NS
NBUF = 3
NI = 6

_mesh = functools.partial(
    plsc.VectorSubcoreMesh, core_axis_name="c", subcore_axis_name="s")


GIDS = IDP // NS


@functools.partial(
    pl.kernel,
    out_type=(jax.ShapeDtypeStruct((NC, NPAD, D), jnp.float32),
              jax.ShapeDtypeStruct((NC, IDP, D), jnp.float32)),
    mesh=_mesh(),
    scratch_types=[
        pltpu.VMEM((NI, 2, CH), jnp.int32),
        pltpu.VMEM((NBUF, CH, D), jnp.float32),
        pltpu.VMEM_SHARED((NPAD, D), jnp.float32),
        [pltpu.SemaphoreType.DMA] * NI,
        [pltpu.SemaphoreType.DMA] * NBUF,
        [pltpu.SemaphoreType.DMA] * NBUF,
    ],
)
def _edge_kernel(x_hbm, src_hbm, dst_hbm, zeros_hbm, ids_hbm,
                 out_hbm, hid_hbm, idx_v, rows_v, agg_sp, isem, gsem, ssem):
    c = lax.axis_index("c")
    s = lax.axis_index("s")
    wid = s * NC + c
    r0 = s * RPT

    @pl.when(c == 0)
    def _():
        @pl.when(s < NS - 1)
        def _():
            pltpu.sync_copy(x_hbm.at[pl.ds(r0, RPT)],
                            agg_sp.at[pl.ds(r0, RPT)])

        @pl.when(s == NS - 1)
        def _():
            lo = (NS - 1) * RPT
            pltpu.sync_copy(x_hbm.at[pl.ds(lo, N - lo)],
                            agg_sp.at[pl.ds(lo, N - lo)])
            pltpu.sync_copy(zeros_hbm.at[pl.ds(N, NPAD - N)],
                            agg_sp.at[pl.ds(N, NPAD - N)])

    @pl.when(c != 0)
    def _():
        pltpu.sync_copy(zeros_hbm.at[pl.ds(r0, RPT)],
                        agg_sp.at[pl.ds(r0, RPT)])

    id_ = [None] * NCHP
    gd = [None] * NCHP
    sd = [None] * NCHP

    ebase = wid * EPW

    def _istart(t):
        n = CH if t < NCHP - 1 else TAIL
        d1 = pltpu.async_copy(
            src_hbm.at[pl.ds(ebase + t * CH, n)],
            idx_v.at[t % NI].at[0].at[pl.ds(0, n)], isem[t % NI])
        d2 = pltpu.async_copy(
            dst_hbm.at[pl.ds(ebase + t * CH, n)],
            idx_v.at[t % NI].at[1].at[pl.ds(0, n)], isem[t % NI])
        id_[t] = (d1, d2)

    def _scatter(q):
        gd[q].wait()
        sd[q] = pltpu.async_copy(
            rows_v.at[q % NBUF], agg_sp.at[idx_v.at[q % NI].at[1]],
            ssem[q % NBUF], add=True)

    for t in range(NI):
        _istart(t)
    plsc.subcore_barrier()
    for j in range(NCHP):
        b = j % NBUF
        if j >= NBUF:
            sd[j - NBUF].wait()
            if j - NBUF + NI < NCHP:
                _istart(j - NBUF + NI)
        id_[j][0].wait()
        id_[j][1].wait()
        if j == NCHP - 1:
            slot = idx_v.at[j % NI]
            for k in range(TAIL // L, CH // L):
                v = lax.iota(jnp.int32, L) + (k * L)
                slot[0, pl.ds(k * L, L)] = v
                slot[1, pl.ds(k * L, L)] = v + N
        gd[j] = pltpu.async_copy(
            x_hbm.at[idx_v.at[j % NI].at[0]], rows_v.at[b], gsem[b])
        if j - (NBUF - 1) >= 0:
            _scatter(j - (NBUF - 1))
    for q in range(NCHP - (NBUF - 1), NCHP):
        _scatter(q)
    for j in range(NCHP - NBUF, NCHP):
        sd[j].wait()
    plsc.subcore_barrier()
    wb = pltpu.async_copy(agg_sp.at[pl.ds(r0, RPT)],
                          out_hbm.at[c].at[pl.ds(r0, RPT)], ssem[0])
    ib = s * GIDS
    ids_slot = idx_v.at[0].at[0].at[pl.ds(0, GIDS)]
    grows = rows_v.at[0].at[pl.ds(0, GIDS)]
    pltpu.sync_copy(ids_hbm.at[pl.ds(ib, GIDS)], ids_slot)
    pltpu.async_copy(agg_sp.at[ids_slot], grows, gsem[0]).wait()
    pltpu.sync_copy(grows, hid_hbm.at[c].at[pl.ds(ib, GIDS)])
    wb.wait()


BLK = 512
GMAIN = NPAD // BLK
GRID2 = GMAIN + IDP // BLK


def _tc_body(p0_ref, p1_ref, h0_ref, h1_ref, w1_ref, b1_ref, w2_ref, b2_ref,
             wi1_ref, bi1_ref, wi2_ref, bi2_ref, o_ref, y_ref):
    i = pl.program_id(0)
    main = i < GMAIN
    h = jnp.where(main, p0_ref[...] + p1_ref[...], h0_ref[...] + h1_ref[...])
    wa = jnp.where(main, w1_ref[...], wi1_ref[...])
    ba = jnp.where(main, b1_ref[...], bi1_ref[...])
    wb = jnp.where(main, w2_ref[...], wi2_ref[...])
    bb = jnp.where(main, b2_ref[...], bi2_ref[...])
    a = jnp.maximum(jnp.dot(h, wa, preferred_element_type=jnp.float32) + ba,
                    0.0)
    o = jnp.dot(a, wb, preferred_element_type=jnp.float32) + bb

    @pl.when(main)
    def _():
        o_ref[...] = o

    @pl.when(jnp.logical_not(main))
    def _():
        y_ref[...] = o


def _tc_mlps(p0, p1, h0, h1, w1t, b1, w2t, b2, wi1t, bi1, wi2t, bi2):
    p_spec = pl.BlockSpec((BLK, D), lambda i: (jnp.minimum(i, GMAIN - 1), 0))
    h_spec = pl.BlockSpec((BLK, D), lambda i: (jnp.maximum(i - GMAIN, 0), 0))
    full_spec = pl.BlockSpec((D, D), lambda i: (0, 0))
    bias_spec = pl.BlockSpec((1, D), lambda i: (0, 0))
    return pl.pallas_call(
        _tc_body,
        grid=(GRID2,),
        in_specs=[p_spec, p_spec, h_spec, h_spec,
                  full_spec, bias_spec, full_spec, bias_spec,
                  full_spec, bias_spec, full_spec, bias_spec],
        out_specs=[p_spec, h_spec],
        out_shape=[jax.ShapeDtypeStruct((NPAD, D), jnp.float32),
                   jax.ShapeDtypeStruct((IDP, D), jnp.float32)],
    )(p0, p1, h0, h1, w1t, b1, w2t, b2, wi1t, bi1, wi2t, bi2)


@functools.partial(
    pl.kernel,
    out_type=jax.ShapeDtypeStruct((N, D), jnp.float32),
    mesh=_mesh(),
    scratch_types=[
        pltpu.VMEM((CROWS,), jnp.int32),
        pltpu.VMEM((1, CROWS), jnp.int32),
        pltpu.VMEM((CROWS, D), jnp.float32),
        pltpu.VMEM_SHARED((SPE, D), jnp.float32),
    ],
)
def _scatter_kernel(out_main_hbm, ids_hbm, yid_hbm, final_hbm,
                    ids_v, idx2_v, yrows_v, outsp):
    c = lax.axis_index("c")
    s = lax.axis_index("s")
    base_row = c * HALF

    pltpu.sync_copy(out_main_hbm.at[pl.ds(base_row + s * RPT_E, RPT_E)],
                    outsp.at[pl.ds(s * RPT_E, RPT_E)])

    ib = s * CROWS
    pltpu.sync_copy(ids_hbm.at[pl.ds(ib, CROWS)], ids_v)
    for k in range(CROWS // L):
        iv = ids_v[pl.ds(k * L, L)]
        lv = iv - base_row
        oor = (lv < 0) | (lv >= HALF)
        tv = HALF + lax.rem(iv, TR_E)
        idx2_v[0, pl.ds(k * L, L)] = jnp.where(oor, tv, lv)
    pltpu.sync_copy(yid_hbm.at[pl.ds(ib, CROWS)], yrows_v)
    plsc.subcore_barrier()
    pltpu.sync_copy(yrows_v, outsp.at[idx2_v.at[0]], add=True)
    plsc.subcore_barrier()

    @pl.when(c == 0)
    def _():
        pltpu.sync_copy(outsp.at[pl.ds(s * WB0, WB0)],
                        final_hbm.at[pl.ds(s * WB0, WB0)])

    @pl.when((c != 0) & (s < NS - 1))
    def _():
        pltpu.sync_copy(outsp.at[pl.ds(s * WB1, WB1)],
                        final_hbm.at[pl.ds(HALF + s * WB1, WB1)])

    @pl.when((c != 0) & (s == NS - 1))
    def _():
        lo = (NS - 1) * WB1
        pltpu.sync_copy(outsp.at[pl.ds(lo, WB1L)],
                        final_hbm.at[pl.ds(HALF + lo, WB1L)])


def kernel(x, edge_index, node_id, W1, b1, W2, b2, Wi1, bi1, Wi2, bi2):
    zeros = jnp.zeros((NPAD, D), jnp.float32)
    src_a = edge_index[0]
    dst_a = jnp.where(src_a == edge_index[1],
                      N + edge_index[1] % TRASH_N, edge_index[1])
    idp = jnp.arange(IDP - node_id.shape[0], dtype=jnp.int32)
    ids = jnp.concatenate([node_id, N + idp % TRASH_N])
    p, hid = _edge_kernel(x, src_a, dst_a, zeros, ids)
    out_main, yid = _tc_mlps(p[0], p[1], hid[0], hid[1],
                             W1.T, b1[None, :], W2.T, b2[None, :],
                             Wi1.T, bi1[None, :], Wi2.T, bi2[None, :])
    return _scatter_kernel(out_main, ids, yid)


# speedup vs baseline: 1.0486x; 1.0379x over previous
import functools

import jax
import jax.numpy as jnp
from jax import lax
from jax.experimental import pallas as pl
from jax.experimental.pallas import tpu as pltpu
from jax.experimental.pallas import tpu_sc as plsc

N, E, D = 10000, 320000, 128
NC, NS, L = 2, 16, 16
NW = NC * NS
CH = 112
NCHP = 90
EPW = E // NW
TAIL = EPW - (NCHP - 1) * CH
NPAD = 10240
TRASH_N = 224
RPT = NPAD // NS
IDP = 1024
NBUF = 3
NI = 6

_mesh = functools.partial(
    plsc.VectorSubcoreMesh, core_axis_name="c", subcore_axis_name="s")


GIDS = IDP // NS


@functools.partial(
    pl.kernel,
    out_type=(jax.ShapeDtypeStruct((NC, NPAD, D), jnp.float32),
              jax.ShapeDtypeStruct((NC, IDP, D), jnp.float32)),
    mesh=_mesh(),
    scratch_types=[
        pltpu.VMEM((NI, 2, CH), jnp.int32),
        pltpu.VMEM((NBUF, CH, D), jnp.float32),
        pltpu.VMEM_SHARED((NPAD, D), jnp.float32),
        [pltpu.SemaphoreType.DMA] * NI,
        [pltpu.SemaphoreType.DMA] * NBUF,
        [pltpu.SemaphoreType.DMA] * NBUF,
    ],
)
def _edge_kernel(x_hbm, src_hbm, dst_hbm, zeros_hbm, ids_hbm,
                 out_hbm, hid_hbm, idx_v, rows_v, agg_sp, isem, gsem, ssem):
    c = lax.axis_index("c")
    s = lax.axis_index("s")
    wid = s * NC + c
    r0 = s * RPT

    @pl.when(c == 0)
    def _():
        @pl.when(s < NS - 1)
        def _():
            pltpu.sync_copy(x_hbm.at[pl.ds(r0, RPT)],
                            agg_sp.at[pl.ds(r0, RPT)])

        @pl.when(s == NS - 1)
        def _():
            lo = (NS - 1) * RPT
            pltpu.sync_copy(x_hbm.at[pl.ds(lo, N - lo)],
                            agg_sp.at[pl.ds(lo, N - lo)])
            pltpu.sync_copy(zeros_hbm.at[pl.ds(N, NPAD - N)],
                            agg_sp.at[pl.ds(N, NPAD - N)])

    @pl.when(c != 0)
    def _():
        pltpu.sync_copy(zeros_hbm.at[pl.ds(r0, RPT)],
                        agg_sp.at[pl.ds(r0, RPT)])

    id_ = [None] * NCHP
    gd = [None] * NCHP
    sd = [None] * NCHP

    ebase = wid * EPW

    def _istart(t):
        n = CH if t < NCHP - 1 else TAIL
        d1 = pltpu.async_copy(
            src_hbm.at[pl.ds(ebase + t * CH, n)],
            idx_v.at[t % NI].at[0].at[pl.ds(0, n)], isem[t % NI])
        d2 = pltpu.async_copy(
            dst_hbm.at[pl.ds(ebase + t * CH, n)],
            idx_v.at[t % NI].at[1].at[pl.ds(0, n)], isem[t % NI])
        id_[t] = (d1, d2)

    def _scatter(q):
        gd[q].wait()
        sd[q] = pltpu.async_copy(
            rows_v.at[q % NBUF], agg_sp.at[idx_v.at[q % NI].at[1]],
            ssem[q % NBUF], add=True)

    for t in range(NI):
        _istart(t)
    plsc.subcore_barrier()
    for j in range(NCHP):
        b = j % NBUF
        if j >= NBUF:
            sd[j - NBUF].wait()
            if j - NBUF + NI < NCHP:
                _istart(j - NBUF + NI)
        id_[j][0].wait()
        id_[j][1].wait()
        if j == NCHP - 1:
            slot = idx_v.at[j % NI]
            for k in range(TAIL // L, CH // L):
                v = lax.iota(jnp.int32, L) + (k * L)
                slot[0, pl.ds(k * L, L)] = v
                slot[1, pl.ds(k * L, L)] = v + N
        gd[j] = pltpu.async_copy(
            x_hbm.at[idx_v.at[j % NI].at[0]], rows_v.at[b], gsem[b])
        if j - (NBUF - 1) >= 0:
            _scatter(j - (NBUF - 1))
    for q in range(NCHP - (NBUF - 1), NCHP):
        _scatter(q)
    for j in range(NCHP - NBUF, NCHP):
        sd[j].wait()
    plsc.subcore_barrier()
    wb = pltpu.async_copy(agg_sp.at[pl.ds(r0, RPT)],
                          out_hbm.at[c].at[pl.ds(r0, RPT)], ssem[0])
    ib = s * GIDS
    ids_slot = idx_v.at[0].at[0].at[pl.ds(0, GIDS)]
    grows = rows_v.at[0].at[pl.ds(0, GIDS)]
    pltpu.sync_copy(ids_hbm.at[pl.ds(ib, GIDS)], ids_slot)
    pltpu.async_copy(agg_sp.at[ids_slot], grows, gsem[0]).wait()
    pltpu.sync_copy(grows, hid_hbm.at[c].at[pl.ds(ib, GIDS)])
    wb.wait()


IBLK = 512
NIB = IDP // IBLK
MBLK = 400
GRID2 = NIB + N // MBLK


def _tc_body(ids_ref, p0_ref, p1_ref, h0_ref, h1_ref,
             w1_ref, b1_ref, w2_ref, b2_ref,
             wi1_ref, bi1_ref, wi2_ref, bi2_ref, o_ref, yid_scr):
    i = pl.program_id(0)

    @pl.when(i < NIB)
    def _():
        h = h0_ref[...] + h1_ref[...]
        a = jnp.maximum(
            jnp.dot(h, wi1_ref[...], preferred_element_type=jnp.float32)
            + bi1_ref[...], 0.0)
        y = (jnp.dot(a, wi2_ref[...], preferred_element_type=jnp.float32)
             + bi2_ref[...])
        yid_scr[pl.ds(pl.multiple_of(i * IBLK, IBLK), IBLK), :] = y

    @pl.when(i >= NIB)
    def _():
        h = p0_ref[...] + p1_ref[...]
        a = jnp.maximum(
            jnp.dot(h, w1_ref[...], preferred_element_type=jnp.float32)
            + b1_ref[...], 0.0)
        o = (jnp.dot(a, w2_ref[...], preferred_element_type=jnp.float32)
             + b2_ref[...])
        base = (i - NIB) * MBLK
        rows = base + jax.lax.broadcasted_iota(jnp.int32, (MBLK, IDP), 0)
        pmat = (rows == ids_ref[...][None, :]).astype(jnp.float32)
        o_ref[...] = o + jnp.dot(pmat, yid_scr[...],
                                 preferred_element_type=jnp.float32)


def _tc_mlps(ids, p0, p1, h0, h1, w1t, b1, w2t, b2, wi1t, bi1, wi2t, bi2):
    p_spec = pl.BlockSpec((MBLK, D), lambda i: (jnp.maximum(i - NIB, 0), 0))
    h_spec = pl.BlockSpec((IBLK, D), lambda i: (jnp.minimum(i, NIB - 1), 0))
    ids_spec = pl.BlockSpec((IDP,), lambda i: (0,))
    full_spec = pl.BlockSpec((D, D), lambda i: (0, 0))
    bias_spec = pl.BlockSpec((1, D), lambda i: (0, 0))
    return pl.pallas_call(
        _tc_body,
        grid=(GRID2,),
        in_specs=[ids_spec, p_spec, p_spec, h_spec, h_spec,
                  full_spec, bias_spec, full_spec, bias_spec,
                  full_spec, bias_spec, full_spec, bias_spec],
        out_specs=p_spec,
        out_shape=jax.ShapeDtypeStruct((N, D), jnp.float32),
        scratch_shapes=[pltpu.VMEM((IDP, D), jnp.float32)],
    )(ids, p0, p1, h0, h1, w1t, b1, w2t, b2, wi1t, bi1, wi2t, bi2)


def kernel(x, edge_index, node_id, W1, b1, W2, b2, Wi1, bi1, Wi2, bi2):
    zeros = jnp.zeros((NPAD, D), jnp.float32)
    src_a = edge_index[0]
    dst_a = jnp.where(src_a == edge_index[1],
                      N + edge_index[1] % TRASH_N, edge_index[1])
    idp = jnp.arange(IDP - node_id.shape[0], dtype=jnp.int32)
    ids = jnp.concatenate([node_id, N + idp % TRASH_N])
    p, hid = _edge_kernel(x, src_a, dst_a, zeros, ids)
    return _tc_mlps(ids, p[0], p[1], hid[0], hid[1],
                    W1.T, b1[None, :], W2.T, b2[None, :],
                    Wi1.T, bi1[None, :], Wi2.T, bi2[None, :])
